# baseline (speedup 1.0000x reference)
import functools

import jax
import jax.numpy as jnp
from jax import lax
from jax.experimental import pallas as pl
from jax.experimental.pallas import tpu as pltpu
from jax.experimental.pallas import tpu_sc as plsc

_LANES = 16


@functools.lru_cache(maxsize=None)
def _make_sc_pool(vocab, emb, batch, seqlen, nc, ns):
    nw = nc * ns
    assert batch % nw == 0 and seqlen % 2 == 0 and emb % _LANES == 0
    s_per = batch // nw
    half = seqlen // 2
    rows_per = s_per * 2
    nvec = emb // _LANES
    unroll = 10
    assert half % unroll == 0
    mesh = plsc.VectorSubcoreMesh(core_axis_name="c", subcore_axis_name="s")

    @functools.partial(
        pl.kernel,
        out_type=jax.ShapeDtypeStruct((batch, emb), jnp.float32),
        mesh=mesh,
        scratch_types=[
            pltpu.VMEM((rows_per, half), jnp.int32),
            pltpu.VMEM((2, half, emb), jnp.float32),
            pltpu.VMEM((s_per, emb), jnp.float32),
            pltpu.SemaphoreType.DMA,
            pltpu.SemaphoreType.DMA,
        ],
        compiler_params=pltpu.CompilerParams(use_tc_tiling_on_sc=False),
    )
    def sc_pool(idx_hbm, table_hbm, sums_hbm, idx_v, rows_v, sums_v, sem0, sem1):
        wid = lax.axis_index("s") * nc + lax.axis_index("c")
        row_base = wid * rows_per
        pltpu.sync_copy(idx_hbm.at[pl.ds(row_base, rows_per)], idx_v)
        rows0 = rows_v.at[0]
        rows1 = rows_v.at[1]

        def fire(j, dst, sem):
            pltpu.async_copy(table_hbm.at[idx_v.at[j]], dst, sem)

        def wait(j, dst, sem):
            pltpu.make_async_copy(table_hbm.at[idx_v.at[j]], dst, sem).wait()

        def accum(rows, acc):
            def body(i, carry):
                carry = list(carry)
                r = i * unroll
                for k in range(unroll):
                    off = (k % 2) * nvec
                    for v in range(nvec):
                        carry[off + v] += rows[r + k, pl.ds(v * _LANES, _LANES)]
                return tuple(carry)

            return lax.fori_loop(0, half // unroll, body, acc)

        zero = jnp.zeros((_LANES,), jnp.float32)
        fire(0, rows0, sem0)

        def sentence(s, carry):
            j0 = 2 * s
            fire(j0 + 1, rows1, sem1)
            wait(j0, rows0, sem0)
            acc = accum(rows0, (zero,) * (2 * nvec))

            @pl.when(s < s_per - 1)
            def _():
                fire(j0 + 2, rows0, sem0)

            wait(j0 + 1, rows1, sem1)
            acc = accum(rows1, acc)
            for v in range(nvec):
                sums_v[s, pl.ds(v * _LANES, _LANES)] = acc[v] + acc[nvec + v]
            return carry

        lax.fori_loop(0, s_per, sentence, 0)
        pltpu.sync_copy(sums_v, sums_hbm.at[pl.ds(wid * s_per, s_per)])

    return sc_pool


@functools.lru_cache(maxsize=None)
def _make_tc_head(batch, seqlen, emb, nclass, pad):
    bb = 512
    assert batch % bb == 0

    def body(sent_ref, sums_ref, w_ref, b_ref, out_ref):
        cnt = jnp.sum((sent_ref[...] != pad).astype(jnp.float32), axis=1,
                      keepdims=True)
        pooled = sums_ref[...] / cnt
        logits = lax.dot_general(pooled, w_ref[...], (((1,), (1,)), ((), ())),
                                 preferred_element_type=jnp.float32)
        logits = logits + b_ref[...]
        shifted = logits - jnp.max(logits, axis=1, keepdims=True)
        lse = jnp.log(jnp.sum(jnp.exp(shifted), axis=1, keepdims=True))
        out_ref[...] = shifted - lse

    return pl.pallas_call(
        body,
        grid=(batch // bb,),
        in_specs=[
            pl.BlockSpec((bb, seqlen), lambda i: (i, 0)),
            pl.BlockSpec((bb, emb), lambda i: (i, 0)),
            pl.BlockSpec((nclass, emb), lambda i: (0, 0)),
            pl.BlockSpec((1, nclass), lambda i: (0, 0)),
        ],
        out_specs=pl.BlockSpec((bb, nclass), lambda i: (i, 0)),
        out_shape=jax.ShapeDtypeStruct((batch, nclass), jnp.float32),
    )


def kernel(sentences, emb_table, fc_w, fc_b):
    batch, seqlen = sentences.shape
    vocab, emb = emb_table.shape
    nclass = fc_w.shape[0]
    info = plsc.get_sparse_core_info()
    nc, ns = info.num_cores, info.num_subcores
    sent_i32 = sentences.astype(jnp.int32)
    idx_flat = lax.optimization_barrier(
        sent_i32.reshape(batch * seqlen // 128, 128))
    idx2 = idx_flat.reshape(batch * 2, seqlen // 2)
    packed = lax.optimization_barrier(
        emb_table.reshape(vocab * emb // 128, 128))
    table_lin = packed.reshape(vocab, emb)
    sums = _make_sc_pool(vocab, emb, batch, seqlen, nc, ns)(idx2, table_lin)
    head = _make_tc_head(batch, seqlen, emb, nclass, 0)
    return head(sent_i32, sums, fc_w, fc_b.reshape(1, nclass))


# speedup vs baseline: 1.4610x; 1.4610x over previous
import functools

import jax
import jax.numpy as jnp
from jax import lax
from jax.experimental import pallas as pl
from jax.experimental.pallas import tpu as pltpu
from jax.experimental.pallas import tpu_sc as plsc

_LANES = 16


@functools.lru_cache(maxsize=None)
def _make_sc_pool(vocab, emb, batch, seqlen, nc, ns):
    nw = nc * ns
    assert batch % nw == 0 and seqlen % 2 == 0 and emb % _LANES == 0
    s_per = batch // nw
    half = seqlen // 2
    rows_per = s_per * 2
    nvec = emb // _LANES
    unroll = 10
    assert half % unroll == 0
    mesh = plsc.VectorSubcoreMesh(core_axis_name="c", subcore_axis_name="s")

    @functools.partial(
        pl.kernel,
        out_type=jax.ShapeDtypeStruct((batch, emb), jnp.float32),
        mesh=mesh,
        scratch_types=[
            pltpu.VMEM((rows_per, half), jnp.int32),
            pltpu.VMEM((2, half, emb), jnp.float32),
            pltpu.VMEM((s_per, emb), jnp.float32),
            pltpu.SemaphoreType.DMA,
            pltpu.SemaphoreType.DMA,
        ],
        compiler_params=pltpu.CompilerParams(use_tc_tiling_on_sc=False),
    )
    def sc_pool(idx_hbm, table_hbm, sums_hbm, idx_v, rows_v, sums_v, sem0, sem1):
        wid = lax.axis_index("s") * nc + lax.axis_index("c")
        row_base = wid * rows_per
        pltpu.sync_copy(idx_hbm.at[pl.ds(row_base, rows_per)], idx_v)
        rows0 = rows_v.at[0]
        rows1 = rows_v.at[1]

        def fire(j, dst, sem):
            pltpu.async_copy(table_hbm.at[idx_v.at[j]], dst, sem)

        def wait(j, dst, sem):
            pltpu.make_async_copy(table_hbm.at[idx_v.at[j]], dst, sem).wait()

        def accum(rows, acc):
            def body(i, carry):
                carry = list(carry)
                r = i * unroll
                for k in range(unroll):
                    off = (k % 2) * nvec
                    for v in range(nvec):
                        carry[off + v] += rows[r + k, pl.ds(v * _LANES, _LANES)]
                return tuple(carry)

            return lax.fori_loop(0, half // unroll, body, acc)

        zero = jnp.zeros((_LANES,), jnp.float32)
        fire(0, rows0, sem0)

        def sentence(s, carry):
            j0 = 2 * s
            fire(j0 + 1, rows1, sem1)
            wait(j0, rows0, sem0)
            acc = accum(rows0, (zero,) * (2 * nvec))

            @pl.when(s < s_per - 1)
            def _():
                fire(j0 + 2, rows0, sem0)

            wait(j0 + 1, rows1, sem1)
            acc = accum(rows1, acc)
            for v in range(nvec):
                sums_v[s, pl.ds(v * _LANES, _LANES)] = acc[v] + acc[nvec + v]
            return carry

        lax.fori_loop(0, s_per, sentence, 0)
        pltpu.sync_copy(sums_v, sums_hbm.at[pl.ds(wid * s_per, s_per)])

    return sc_pool


@functools.lru_cache(maxsize=None)
def _make_tc_packer(vocab, emb, cb):
    nblk = (vocab + cb - 1) // cb
    half = cb // 2
    rows_out = cb * emb // 128

    def body(t_ref, out_ref):
        xt = t_ref[...].T
        out_ref[:, 0:emb] = xt[0:half]
        out_ref[:, emb:2 * emb] = xt[half:cb]

    return pl.pallas_call(
        body,
        grid=(nblk,),
        in_specs=[pl.BlockSpec((emb, cb), lambda i: (0, i))],
        out_specs=pl.BlockSpec((rows_out, 128), lambda i: (i, 0)),
        out_shape=jax.ShapeDtypeStruct((nblk * rows_out, 128), jnp.float32),
    )


@functools.lru_cache(maxsize=None)
def _make_tc_head(batch, seqlen, emb, nclass, pad):
    bb = 512
    assert batch % bb == 0

    def body(sent_ref, sums_ref, w_ref, b_ref, out_ref):
        cnt = jnp.sum((sent_ref[...] != pad).astype(jnp.float32), axis=1,
                      keepdims=True)
        pooled = sums_ref[...] / cnt
        logits = lax.dot_general(pooled, w_ref[...], (((1,), (1,)), ((), ())),
                                 preferred_element_type=jnp.float32)
        logits = logits + b_ref[...]
        shifted = logits - jnp.max(logits, axis=1, keepdims=True)
        lse = jnp.log(jnp.sum(jnp.exp(shifted), axis=1, keepdims=True))
        out_ref[...] = shifted - lse

    return pl.pallas_call(
        body,
        grid=(batch // bb,),
        in_specs=[
            pl.BlockSpec((bb, seqlen), lambda i: (i, 0)),
            pl.BlockSpec((bb, emb), lambda i: (i, 0)),
            pl.BlockSpec((nclass, emb), lambda i: (0, 0)),
            pl.BlockSpec((1, nclass), lambda i: (0, 0)),
        ],
        out_specs=pl.BlockSpec((bb, nclass), lambda i: (i, 0)),
        out_shape=jax.ShapeDtypeStruct((batch, nclass), jnp.float32),
    )


def kernel(sentences, emb_table, fc_w, fc_b):
    batch, seqlen = sentences.shape
    vocab, emb = emb_table.shape
    nclass = fc_w.shape[0]
    info = plsc.get_sparse_core_info()
    nc, ns = info.num_cores, info.num_subcores
    sent_i32 = sentences.astype(jnp.int32)
    cb = 4096
    half = cb // 2
    nblk = (vocab + cb - 1) // cb
    vocab_pad = nblk * cb
    packed = _make_tc_packer(vocab, emb, cb)(emb_table.T)
    table_lin = packed.reshape(vocab_pad, emb)
    lo = sent_i32 % cb
    idx_t = (sent_i32 - lo) + 2 * (lo % half) + lo // half
    idx_flat = lax.optimization_barrier(
        idx_t.reshape(batch * seqlen // 128, 128))
    idx2 = idx_flat.reshape(batch * 2, seqlen // 2)
    sums = _make_sc_pool(vocab_pad, emb, batch, seqlen, nc, ns)(idx2, table_lin)
    head = _make_tc_head(batch, seqlen, emb, nclass, 0)
    return head(sent_i32, sums, fc_w, fc_b.reshape(1, nclass))


# speedup vs baseline: 1.5709x; 1.0752x over previous
import functools

import jax
import jax.numpy as jnp
from jax import lax
from jax.experimental import pallas as pl
from jax.experimental.pallas import tpu as pltpu
from jax.experimental.pallas import tpu_sc as plsc

_LANES = 16
_HI = -65536


def _round_bf16_hi(x):
    b = lax.bitcast_convert_type(x, jnp.int32)
    return b + 0x7FFF + (lax.shift_right_logical(b, 16) & 1)


@functools.lru_cache(maxsize=None)
def _make_tc_packer(vocab, emb, cb):
    assert emb == 64
    nblk = (vocab + cb - 1) // cb
    q = cb // 4

    def body(t_ref, out_ref):
        x = t_ref[...]
        hi = _round_bf16_hi(x[0:32, :]) & _HI
        lo = lax.shift_right_logical(_round_bf16_hi(x[32:64, :]), 16)
        wt = (hi | lo).T
        out_ref[:, 0:32] = wt[0:q]
        out_ref[:, 32:64] = wt[q:2 * q]
        out_ref[:, 64:96] = wt[2 * q:3 * q]
        out_ref[:, 96:128] = wt[3 * q:4 * q]

    return pl.pallas_call(
        body,
        grid=(nblk,),
        in_specs=[pl.BlockSpec((emb, cb), lambda i: (0, i))],
        out_specs=pl.BlockSpec((q, 128), lambda i: (i, 0)),
        out_shape=jax.ShapeDtypeStruct((nblk * q, 128), jnp.int32),
    )


@functools.lru_cache(maxsize=None)
def _make_sc_pool(vocab_pad, emb, batch, seqlen, nc, ns):
    nw = nc * ns
    assert batch % nw == 0 and seqlen % 2 == 0 and emb == 64
    s_per = batch // nw
    half = seqlen // 2
    rows_per = s_per * 2
    words = emb // 2
    unroll = 10
    assert half % unroll == 0
    mesh = plsc.VectorSubcoreMesh(core_axis_name="c", subcore_axis_name="s")

    @functools.partial(
        pl.kernel,
        out_type=jax.ShapeDtypeStruct((batch, emb), jnp.float32),
        mesh=mesh,
        scratch_types=[
            pltpu.VMEM((rows_per, half), jnp.int32),
            pltpu.VMEM((2, half, words), jnp.int32),
            pltpu.VMEM((s_per, emb), jnp.float32),
            pltpu.SemaphoreType.DMA,
            pltpu.SemaphoreType.DMA,
        ],
        compiler_params=pltpu.CompilerParams(use_tc_tiling_on_sc=False,
                                             needs_layout_passes=False),
    )
    def sc_pool(idx_hbm, table_hbm, sums_hbm, idx_v, rows_v, sums_v, sem0, sem1):
        wid = lax.axis_index("s") * nc + lax.axis_index("c")
        row_base = wid * rows_per
        pltpu.sync_copy(idx_hbm.at[pl.ds(row_base, rows_per)], idx_v)
        rows0 = rows_v.at[0]
        rows1 = rows_v.at[1]

        def fire(j, dst, sem):
            pltpu.async_copy(table_hbm.at[idx_v.at[j]], dst, sem)

        def wait(j, dst, sem):
            pltpu.make_async_copy(table_hbm.at[idx_v.at[j]], dst, sem).wait()

        def accum(rows, acc):
            def body(i, carry):
                carry = list(carry)
                r = i * unroll
                for k in range(unroll):
                    off = (k % 2) * 4
                    w0 = rows[r + k, pl.ds(0, _LANES)]
                    w1 = rows[r + k, pl.ds(_LANES, _LANES)]
                    carry[off + 0] += plsc.bitcast(w0 & _HI, jnp.float32)
                    carry[off + 1] += plsc.bitcast(w1 & _HI, jnp.float32)
                    carry[off + 2] += plsc.bitcast(
                        lax.shift_left(w0, 16), jnp.float32)
                    carry[off + 3] += plsc.bitcast(
                        lax.shift_left(w1, 16), jnp.float32)
                return tuple(carry)

            return lax.fori_loop(0, half // unroll, body, acc)

        zero = jnp.zeros((_LANES,), jnp.float32)
        fire(0, rows0, sem0)

        def sentence(s, carry):
            j0 = 2 * s
            fire(j0 + 1, rows1, sem1)
            wait(j0, rows0, sem0)
            acc = accum(rows0, (zero,) * 8)

            @pl.when(s < s_per - 1)
            def _():
                fire(j0 + 2, rows0, sem0)

            wait(j0 + 1, rows1, sem1)
            acc = accum(rows1, acc)
            sums_v[s, pl.ds(0, _LANES)] = acc[0] + acc[4]
            sums_v[s, pl.ds(_LANES, _LANES)] = acc[1] + acc[5]
            sums_v[s, pl.ds(2 * _LANES, _LANES)] = acc[2] + acc[6]
            sums_v[s, pl.ds(3 * _LANES, _LANES)] = acc[3] + acc[7]
            return carry

        lax.fori_loop(0, s_per, sentence, 0)
        pltpu.sync_copy(sums_v, sums_hbm.at[pl.ds(wid * s_per, s_per)])

    return sc_pool


@functools.lru_cache(maxsize=None)
def _make_tc_head(batch, seqlen, emb, nclass, pad):
    bb = 512
    assert batch % bb == 0

    def body(sent_ref, sums_ref, w_ref, b_ref, out_ref):
        cnt = jnp.sum((sent_ref[...] != pad).astype(jnp.float32), axis=1,
                      keepdims=True)
        pooled = sums_ref[...] / cnt
        logits = lax.dot_general(pooled, w_ref[...], (((1,), (1,)), ((), ())),
                                 preferred_element_type=jnp.float32)
        logits = logits + b_ref[...]
        shifted = logits - jnp.max(logits, axis=1, keepdims=True)
        lse = jnp.log(jnp.sum(jnp.exp(shifted), axis=1, keepdims=True))
        out_ref[...] = shifted - lse

    return pl.pallas_call(
        body,
        grid=(batch // bb,),
        in_specs=[
            pl.BlockSpec((bb, seqlen), lambda i: (i, 0)),
            pl.BlockSpec((bb, emb), lambda i: (i, 0)),
            pl.BlockSpec((nclass, emb), lambda i: (0, 0)),
            pl.BlockSpec((1, nclass), lambda i: (0, 0)),
        ],
        out_specs=pl.BlockSpec((bb, nclass), lambda i: (i, 0)),
        out_shape=jax.ShapeDtypeStruct((batch, nclass), jnp.float32),
    )


def kernel(sentences, emb_table, fc_w, fc_b):
    batch, seqlen = sentences.shape
    vocab, emb = emb_table.shape
    nclass = fc_w.shape[0]
    info = plsc.get_sparse_core_info()
    nc, ns = info.num_cores, info.num_subcores
    sent_i32 = sentences.astype(jnp.int32)
    cb = 4096
    q = cb // 4
    nblk = (vocab + cb - 1) // cb
    vocab_pad = nblk * cb
    packed = _make_tc_packer(vocab, emb, cb)(emb_table.T)
    table_lin = packed.reshape(vocab_pad, emb // 2)
    lo = sent_i32 % cb
    idx_t = (sent_i32 - lo) + 4 * (lo % q) + lo // q
    idx_flat = lax.optimization_barrier(
        idx_t.reshape(batch * seqlen // 128, 128))
    idx2 = idx_flat.reshape(batch * 2, seqlen // 2)
    sums = _make_sc_pool(vocab_pad, emb, batch, seqlen, nc, ns)(idx2, table_lin)
    head = _make_tc_head(batch, seqlen, emb, nclass, 0)
    return head(sent_i32, sums, fc_w, fc_b.reshape(1, nclass))


# speedup vs baseline: 1.9198x; 1.2221x over previous
import functools

import jax
import jax.numpy as jnp
from jax import lax
from jax.experimental import pallas as pl
from jax.experimental.pallas import tpu as pltpu
from jax.experimental.pallas import tpu_sc as plsc

_LANES = 16
_HI = -65536


def _round_bf16_hi(x):
    b = lax.bitcast_convert_type(x, jnp.int32)
    return b + 0x7FFF + (lax.shift_right_logical(b, 16) & 1)


@functools.lru_cache(maxsize=None)
def _make_tc_packer(vocab, emb, cb):
    assert emb == 64
    nblk = (vocab + cb - 1) // cb
    q = cb // 4

    def body(t_ref, out_ref):
        x = t_ref[...]
        hi = _round_bf16_hi(x[0:32, :]) & _HI
        lo = lax.shift_right_logical(_round_bf16_hi(x[32:64, :]), 16)
        w = hi | lo
        wp4 = jnp.concatenate(
            [w[:, 0:q], w[:, q:2 * q], w[:, 2 * q:3 * q], w[:, 3 * q:4 * q]],
            axis=0)
        out_ref[...] = wp4.T

    return pl.pallas_call(
        body,
        grid=(nblk,),
        in_specs=[pl.BlockSpec((emb, cb), lambda i: (0, i))],
        out_specs=pl.BlockSpec((q, 128), lambda i: (i, 0)),
        out_shape=jax.ShapeDtypeStruct((nblk * q, 128), jnp.int32),
    )


@functools.lru_cache(maxsize=None)
def _make_sc_pool(vocab_pad, emb, batch, seqlen, nc, ns):
    nw = nc * ns
    assert batch % nw == 0 and seqlen % 2 == 0 and emb == 64
    s_per = batch // nw
    half = seqlen // 2
    rows_per = s_per * 2
    words = emb // 2
    unroll = 10
    assert half % unroll == 0
    mesh = plsc.VectorSubcoreMesh(core_axis_name="c", subcore_axis_name="s")

    @functools.partial(
        pl.kernel,
        out_type=jax.ShapeDtypeStruct((batch, emb), jnp.float32),
        mesh=mesh,
        scratch_types=[
            pltpu.VMEM((rows_per, half), jnp.int32),
            pltpu.VMEM((2, half, words), jnp.int32),
            pltpu.VMEM((s_per, emb), jnp.float32),
            pltpu.SemaphoreType.DMA,
            pltpu.SemaphoreType.DMA,
        ],
        compiler_params=pltpu.CompilerParams(use_tc_tiling_on_sc=False,
                                             needs_layout_passes=False),
    )
    def sc_pool(idx_hbm, table_hbm, sums_hbm, idx_v, rows_v, sums_v, sem0, sem1):
        wid = lax.axis_index("s") * nc + lax.axis_index("c")
        row_base = wid * rows_per
        pltpu.sync_copy(idx_hbm.at[pl.ds(row_base, rows_per)], idx_v)
        rows0 = rows_v.at[0]
        rows1 = rows_v.at[1]

        def fire(j, dst, sem):
            pltpu.async_copy(table_hbm.at[idx_v.at[j]], dst, sem)

        def wait(j, dst, sem):
            pltpu.make_async_copy(table_hbm.at[idx_v.at[j]], dst, sem).wait()

        def accum(rows, acc):
            def body(i, carry):
                carry = list(carry)
                r = i * unroll
                for k in range(unroll):
                    off = (k % 2) * 4
                    w0 = rows[r + k, pl.ds(0, _LANES)]
                    w1 = rows[r + k, pl.ds(_LANES, _LANES)]
                    carry[off + 0] += plsc.bitcast(w0 & _HI, jnp.float32)
                    carry[off + 1] += plsc.bitcast(w1 & _HI, jnp.float32)
                    carry[off + 2] += plsc.bitcast(
                        lax.shift_left(w0, 16), jnp.float32)
                    carry[off + 3] += plsc.bitcast(
                        lax.shift_left(w1, 16), jnp.float32)
                return tuple(carry)

            return lax.fori_loop(0, half // unroll, body, acc)

        zero = jnp.zeros((_LANES,), jnp.float32)
        fire(0, rows0, sem0)

        def sentence(s, carry):
            j0 = 2 * s
            fire(j0 + 1, rows1, sem1)
            wait(j0, rows0, sem0)
            acc = accum(rows0, (zero,) * 8)

            @pl.when(s < s_per - 1)
            def _():
                fire(j0 + 2, rows0, sem0)

            wait(j0 + 1, rows1, sem1)
            acc = accum(rows1, acc)
            sums_v[s, pl.ds(0, _LANES)] = acc[0] + acc[4]
            sums_v[s, pl.ds(_LANES, _LANES)] = acc[1] + acc[5]
            sums_v[s, pl.ds(2 * _LANES, _LANES)] = acc[2] + acc[6]
            sums_v[s, pl.ds(3 * _LANES, _LANES)] = acc[3] + acc[7]
            return carry

        lax.fori_loop(0, s_per, sentence, 0)
        pltpu.sync_copy(sums_v, sums_hbm.at[pl.ds(wid * s_per, s_per)])

    return sc_pool


@functools.lru_cache(maxsize=None)
def _make_tc_head(batch, seqlen, emb, nclass, pad):
    bb = 512
    assert batch % bb == 0

    def body(sent_ref, sums_ref, w_ref, b_ref, out_ref):
        cnt = jnp.sum((sent_ref[...] != pad).astype(jnp.float32), axis=1,
                      keepdims=True)
        pooled = sums_ref[...] / cnt
        logits = lax.dot_general(pooled, w_ref[...], (((1,), (1,)), ((), ())),
                                 preferred_element_type=jnp.float32)
        logits = logits + b_ref[...]
        shifted = logits - jnp.max(logits, axis=1, keepdims=True)
        lse = jnp.log(jnp.sum(jnp.exp(shifted), axis=1, keepdims=True))
        out_ref[...] = shifted - lse

    return pl.pallas_call(
        body,
        grid=(batch // bb,),
        in_specs=[
            pl.BlockSpec((bb, seqlen), lambda i: (i, 0)),
            pl.BlockSpec((bb, emb), lambda i: (i, 0)),
            pl.BlockSpec((nclass, emb), lambda i: (0, 0)),
            pl.BlockSpec((1, nclass), lambda i: (0, 0)),
        ],
        out_specs=pl.BlockSpec((bb, nclass), lambda i: (i, 0)),
        out_shape=jax.ShapeDtypeStruct((batch, nclass), jnp.float32),
    )


def kernel(sentences, emb_table, fc_w, fc_b):
    batch, seqlen = sentences.shape
    vocab, emb = emb_table.shape
    nclass = fc_w.shape[0]
    info = plsc.get_sparse_core_info()
    nc, ns = info.num_cores, info.num_subcores
    sent_i32 = sentences.astype(jnp.int32)
    cb = 4096
    q = cb // 4
    nblk = (vocab + cb - 1) // cb
    vocab_pad = nblk * cb
    packed = _make_tc_packer(vocab, emb, cb)(emb_table.T)
    table_lin = packed.reshape(vocab_pad, emb // 2)
    lo = sent_i32 % cb
    idx_t = (sent_i32 - lo) + 4 * (lo % q) + lo // q
    idx_flat = lax.optimization_barrier(
        idx_t.reshape(batch * seqlen // 128, 128))
    idx2 = idx_flat.reshape(batch * 2, seqlen // 2)
    sums = _make_sc_pool(vocab_pad, emb, batch, seqlen, nc, ns)(idx2, table_lin)
    head = _make_tc_head(batch, seqlen, emb, nclass, 0)
    return head(sent_i32, sums, fc_w, fc_b.reshape(1, nclass))


# speedup vs baseline: 2.5298x; 1.3177x over previous
import functools

import jax
import jax.numpy as jnp
from jax import lax
from jax.experimental import pallas as pl
from jax.experimental.pallas import tpu as pltpu
from jax.experimental.pallas import tpu_sc as plsc

_LANES = 16
_HI = -65536


def _round_bf16_hi(x):
    b = lax.bitcast_convert_type(x, jnp.int32)
    return b + 0x7FFF + (lax.shift_right_logical(b, 16) & 1)


@functools.lru_cache(maxsize=None)
def _make_tc_packer(vocab, emb, cb):
    assert emb == 64
    nblk = (vocab + cb - 1) // cb
    q = cb // 4

    def body(t_ref, out_ref):
        x = t_ref[...]
        hi = _round_bf16_hi(x[0:32, :]) & _HI
        lo = lax.shift_right_logical(_round_bf16_hi(x[32:64, :]), 16)
        w = hi | lo
        wp4 = jnp.concatenate(
            [w[:, 0:q], w[:, q:2 * q], w[:, 2 * q:3 * q], w[:, 3 * q:4 * q]],
            axis=0)
        out_ref[...] = wp4.T

    return pl.pallas_call(
        body,
        grid=(nblk,),
        in_specs=[pl.BlockSpec((emb, cb), lambda i: (0, i))],
        out_specs=pl.BlockSpec((q, 128), lambda i: (i, 0)),
        out_shape=jax.ShapeDtypeStruct((nblk * q, 128), jnp.int32),
    )


@functools.lru_cache(maxsize=None)
def _make_sc_pool(vocab_pad, emb, batch, seqlen, nc, ns):
    nw = nc * ns
    assert batch % nw == 0 and seqlen % 2 == 0 and emb == 64
    s_per = batch // nw
    half = seqlen // 2
    rows_per = s_per * 2
    words = emb // 2
    unroll = 10
    assert half % unroll == 0
    mesh = plsc.VectorSubcoreMesh(core_axis_name="c", subcore_axis_name="s")

    @functools.partial(
        pl.kernel,
        out_type=jax.ShapeDtypeStruct((batch, emb), jnp.float32),
        mesh=mesh,
        scratch_types=[
            pltpu.VMEM((rows_per, half), jnp.int32),
            pltpu.VMEM((2, half, words), jnp.int32),
            pltpu.VMEM((s_per, emb), jnp.float32),
            pltpu.SemaphoreType.DMA,
            pltpu.SemaphoreType.DMA,
        ],
        compiler_params=pltpu.CompilerParams(use_tc_tiling_on_sc=False,
                                             needs_layout_passes=False),
    )
    def sc_pool(idx_hbm, table_hbm, sums_hbm, idx_v, rows_v, sums_v, sem0, sem1):
        wid = lax.axis_index("s") * nc + lax.axis_index("c")
        row_base = wid * rows_per
        pltpu.sync_copy(idx_hbm.at[pl.ds(row_base, rows_per)], idx_v)
        rows0 = rows_v.at[0]
        rows1 = rows_v.at[1]

        def fire(j, dst, sem):
            pltpu.async_copy(table_hbm.at[idx_v.at[j]], dst, sem)

        def wait(j, dst, sem):
            pltpu.make_async_copy(table_hbm.at[idx_v.at[j]], dst, sem).wait()

        def accum(rows, acc):
            def body(i, carry):
                carry = list(carry)
                r = i * unroll
                for k in range(unroll):
                    off = (k % 2) * 4
                    w0 = rows[r + k, pl.ds(0, _LANES)]
                    w1 = rows[r + k, pl.ds(_LANES, _LANES)]
                    carry[off + 0] += plsc.bitcast(w0 & _HI, jnp.float32)
                    carry[off + 1] += plsc.bitcast(w1 & _HI, jnp.float32)
                    carry[off + 2] += plsc.bitcast(
                        lax.shift_left(w0, 16), jnp.float32)
                    carry[off + 3] += plsc.bitcast(
                        lax.shift_left(w1, 16), jnp.float32)
                return tuple(carry)

            return lax.fori_loop(0, half // unroll, body, acc)

        zero = jnp.zeros((_LANES,), jnp.float32)
        fire(0, rows0, sem0)

        def sentence(s, carry):
            j0 = 2 * s
            fire(j0 + 1, rows1, sem1)
            wait(j0, rows0, sem0)
            acc = accum(rows0, (zero,) * 8)

            @pl.when(s < s_per - 1)
            def _():
                fire(j0 + 2, rows0, sem0)

            wait(j0 + 1, rows1, sem1)
            acc = accum(rows1, acc)
            sums_v[s, pl.ds(0, _LANES)] = acc[0] + acc[4]
            sums_v[s, pl.ds(_LANES, _LANES)] = acc[1] + acc[5]
            sums_v[s, pl.ds(2 * _LANES, _LANES)] = acc[2] + acc[6]
            sums_v[s, pl.ds(3 * _LANES, _LANES)] = acc[3] + acc[7]
            return carry

        lax.fori_loop(0, s_per, sentence, 0)
        pltpu.sync_copy(sums_v, sums_hbm.at[pl.ds(wid * s_per, s_per)])

    return sc_pool


@functools.lru_cache(maxsize=None)
def _make_tc_head(batch, seqlen, emb, nclass, pad):
    bb = 512
    assert batch % bb == 0

    def body(sent_ref, sums_ref, w_ref, b_ref, out_ref):
        cnt = jnp.sum((sent_ref[...] != pad).astype(jnp.float32), axis=1,
                      keepdims=True)
        pooled = sums_ref[...] / cnt
        logits = lax.dot_general(pooled, w_ref[...], (((1,), (1,)), ((), ())),
                                 preferred_element_type=jnp.float32)
        logits = logits + b_ref[...]
        shifted = logits - jnp.max(logits, axis=1, keepdims=True)
        lse = jnp.log(jnp.sum(jnp.exp(shifted), axis=1, keepdims=True))
        out_ref[...] = shifted - lse

    return pl.pallas_call(
        body,
        grid=(batch // bb,),
        in_specs=[
            pl.BlockSpec((bb, seqlen), lambda i: (i, 0)),
            pl.BlockSpec((bb, emb), lambda i: (i, 0)),
            pl.BlockSpec((nclass, emb), lambda i: (0, 0)),
            pl.BlockSpec((1, nclass), lambda i: (0, 0)),
        ],
        out_specs=pl.BlockSpec((bb, nclass), lambda i: (i, 0)),
        out_shape=jax.ShapeDtypeStruct((batch, nclass), jnp.float32),
    )


def kernel(sentences, emb_table, fc_w, fc_b):
    batch, seqlen = sentences.shape
    vocab, emb = emb_table.shape
    nclass = fc_w.shape[0]
    info = plsc.get_sparse_core_info()
    nc, ns = info.num_cores, info.num_subcores
    sent_i32 = sentences.astype(jnp.int32)
    cb = 16384
    q = cb // 4
    nblk = (vocab + cb - 1) // cb
    vocab_pad = nblk * cb
    packed = _make_tc_packer(vocab, emb, cb)(emb_table.T)
    table_lin = packed.reshape(vocab_pad, emb // 2)
    lo = sent_i32 % cb
    idx_t = (sent_i32 - lo) + 4 * (lo % q) + lo // q
    idx_flat = lax.optimization_barrier(
        idx_t.reshape(batch * seqlen // 128, 128))
    idx2 = idx_flat.reshape(batch * 2, seqlen // 2)
    sums = _make_sc_pool(vocab_pad, emb, batch, seqlen, nc, ns)(idx2, table_lin)
    head = _make_tc_head(batch, seqlen, emb, nclass, 0)
    return head(sent_i32, sums, fc_w, fc_b.reshape(1, nclass))


# speedup vs baseline: 3.0855x; 1.2197x over previous
import functools

import jax
import jax.numpy as jnp
from jax import lax
from jax.experimental import pallas as pl
from jax.experimental.pallas import tpu as pltpu
from jax.experimental.pallas import tpu_sc as plsc

_LANES = 16
_HI = -65536


def _round_bf16_hi(x):
    b = lax.bitcast_convert_type(x, jnp.int32)
    return b + 0x7FFF + (lax.shift_right_logical(b, 16) & 1)


@functools.lru_cache(maxsize=None)
def _make_tc_packer(vocab, emb, cb):
    assert emb == 64
    nblk = (vocab + cb - 1) // cb
    q = cb // 4

    def body(t_ref, out_ref):
        x = t_ref[...]
        hi = _round_bf16_hi(x[0:32, :]) & _HI
        lo = lax.shift_right_logical(_round_bf16_hi(x[32:64, :]), 16)
        w = hi | lo
        wp4 = jnp.concatenate(
            [w[:, 0:q], w[:, q:2 * q], w[:, 2 * q:3 * q], w[:, 3 * q:4 * q]],
            axis=0)
        out_ref[...] = wp4.T

    return pl.pallas_call(
        body,
        grid=(nblk,),
        in_specs=[pl.BlockSpec((emb, cb), lambda i: (0, i))],
        out_specs=pl.BlockSpec((q, 128), lambda i: (i, 0)),
        out_shape=jax.ShapeDtypeStruct((nblk * q, 128), jnp.int32),
    )


@functools.lru_cache(maxsize=None)
def _make_sc_pool(vocab_pad, emb, batch, seqlen, nc, ns):
    nw = nc * ns
    assert batch % nw == 0 and seqlen % 2 == 0 and emb == 64
    s_per = batch // nw
    half = seqlen // 2
    rows_per = s_per * 2
    words = emb // 2
    unroll = 10
    assert half % unroll == 0
    mesh = plsc.VectorSubcoreMesh(core_axis_name="c", subcore_axis_name="s")

    @functools.partial(
        pl.kernel,
        out_type=jax.ShapeDtypeStruct((batch, emb), jnp.float32),
        mesh=mesh,
        scratch_types=[
            pltpu.VMEM((rows_per, half), jnp.int32),
            pltpu.VMEM((4, half, words), jnp.int32),
            pltpu.VMEM((s_per, emb), jnp.float32),
            pltpu.SemaphoreType.DMA,
            pltpu.SemaphoreType.DMA,
            pltpu.SemaphoreType.DMA,
            pltpu.SemaphoreType.DMA,
        ],
        compiler_params=pltpu.CompilerParams(use_tc_tiling_on_sc=False,
                                             needs_layout_passes=False),
    )
    def sc_pool(idx_hbm, table_hbm, sums_hbm, idx_v, rows_v, sums_v,
                sem0, sem1, sem2, sem3):
        wid = lax.axis_index("s") * nc + lax.axis_index("c")
        row_base = wid * rows_per
        pltpu.sync_copy(idx_hbm.at[pl.ds(row_base, rows_per)], idx_v)
        slots = [rows_v.at[r] for r in range(4)]
        sems = [sem0, sem1, sem2, sem3]

        def fire(j, dst, sem):
            pltpu.async_copy(table_hbm.at[idx_v.at[j]], dst, sem)

        def wait(j, dst, sem):
            pltpu.make_async_copy(table_hbm.at[idx_v.at[j]], dst, sem).wait()

        def accum(rows, acc):
            def body(i, carry):
                carry = list(carry)
                r = i * unroll
                for k in range(unroll):
                    off = (k % 2) * 4
                    w0 = rows[r + k, pl.ds(0, _LANES)]
                    w1 = rows[r + k, pl.ds(_LANES, _LANES)]
                    carry[off + 0] += plsc.bitcast(w0 & _HI, jnp.float32)
                    carry[off + 1] += plsc.bitcast(w1 & _HI, jnp.float32)
                    carry[off + 2] += plsc.bitcast(
                        lax.shift_left(w0, 16), jnp.float32)
                    carry[off + 3] += plsc.bitcast(
                        lax.shift_left(w1, 16), jnp.float32)
                return tuple(carry)

            return lax.fori_loop(0, half // unroll, body, acc)

        zero = jnp.zeros((_LANES,), jnp.float32)
        for r in range(4):
            fire(r, slots[r], sems[r])

        def store(s, acc):
            sums_v[s, pl.ds(0, _LANES)] = acc[0] + acc[4]
            sums_v[s, pl.ds(_LANES, _LANES)] = acc[1] + acc[5]
            sums_v[s, pl.ds(2 * _LANES, _LANES)] = acc[2] + acc[6]
            sums_v[s, pl.ds(3 * _LANES, _LANES)] = acc[3] + acc[7]

        def pair(i, carry):
            j = 4 * i
            acc = None
            for r in range(4):
                wait(j + r, slots[r], sems[r])
                acc = accum(slots[r], acc if r % 2 else (zero,) * 8)

                @pl.when(i < s_per // 2 - 1)
                def _():
                    fire(j + r + 4, slots[r], sems[r])

                if r % 2:
                    store(2 * i + r // 2, acc)
            return carry

        lax.fori_loop(0, s_per // 2, pair, 0)
        pltpu.sync_copy(sums_v, sums_hbm.at[pl.ds(wid * s_per, s_per)])

    return sc_pool


@functools.lru_cache(maxsize=None)
def _make_tc_head(batch, seqlen, emb, nclass, pad):
    bb = 512
    assert batch % bb == 0

    def body(sent_ref, sums_ref, w_ref, b_ref, out_ref):
        cnt = jnp.sum((sent_ref[...] != pad).astype(jnp.float32), axis=1,
                      keepdims=True)
        pooled = sums_ref[...] / cnt
        logits = lax.dot_general(pooled, w_ref[...], (((1,), (1,)), ((), ())),
                                 preferred_element_type=jnp.float32)
        logits = logits + b_ref[...]
        shifted = logits - jnp.max(logits, axis=1, keepdims=True)
        lse = jnp.log(jnp.sum(jnp.exp(shifted), axis=1, keepdims=True))
        out_ref[...] = shifted - lse

    return pl.pallas_call(
        body,
        grid=(batch // bb,),
        in_specs=[
            pl.BlockSpec((bb, seqlen), lambda i: (i, 0)),
            pl.BlockSpec((bb, emb), lambda i: (i, 0)),
            pl.BlockSpec((nclass, emb), lambda i: (0, 0)),
            pl.BlockSpec((1, nclass), lambda i: (0, 0)),
        ],
        out_specs=pl.BlockSpec((bb, nclass), lambda i: (i, 0)),
        out_shape=jax.ShapeDtypeStruct((batch, nclass), jnp.float32),
    )


def kernel(sentences, emb_table, fc_w, fc_b):
    batch, seqlen = sentences.shape
    vocab, emb = emb_table.shape
    nclass = fc_w.shape[0]
    info = plsc.get_sparse_core_info()
    nc, ns = info.num_cores, info.num_subcores
    sent_i32 = sentences.astype(jnp.int32)
    cb = 16384
    q = cb // 4
    nblk = (vocab + cb - 1) // cb
    vocab_pad = nblk * cb
    packed = _make_tc_packer(vocab, emb, cb)(emb_table.T)
    table_lin = packed.reshape(vocab_pad, emb // 2)
    lo = sent_i32 % cb
    idx_t = (sent_i32 - lo) + 4 * (lo % q) + lo // q
    idx_flat = lax.optimization_barrier(
        idx_t.reshape(batch * seqlen // 128, 128))
    idx2 = idx_flat.reshape(batch * 2, seqlen // 2)
    sums = _make_sc_pool(vocab_pad, emb, batch, seqlen, nc, ns)(idx2, table_lin)
    head = _make_tc_head(batch, seqlen, emb, nclass, 0)
    return head(sent_i32, sums, fc_w, fc_b.reshape(1, nclass))


# speedup vs baseline: 3.3382x; 1.0819x over previous
import functools

import jax
import jax.numpy as jnp
from jax import lax
from jax.experimental import pallas as pl
from jax.experimental.pallas import tpu as pltpu
from jax.experimental.pallas import tpu_sc as plsc

_LANES = 16
_HI = -65536


def _round_bf16_hi(x):
    b = lax.bitcast_convert_type(x, jnp.int32)
    return b + 0x7FFF + (lax.shift_right_logical(b, 16) & 1)


@functools.lru_cache(maxsize=None)
def _make_tc_packer(vocab, emb, cb):
    assert emb == 64
    nblk = (vocab + cb - 1) // cb
    q = cb // 4

    def body(t_ref, out_ref):
        x = t_ref[...]
        hi = _round_bf16_hi(x[0:32, :]) & _HI
        lo = lax.shift_right_logical(_round_bf16_hi(x[32:64, :]), 16)
        w = hi | lo
        wp4 = jnp.concatenate(
            [w[:, 0:q], w[:, q:2 * q], w[:, 2 * q:3 * q], w[:, 3 * q:4 * q]],
            axis=0)
        out_ref[...] = wp4.T

    return pl.pallas_call(
        body,
        grid=(nblk,),
        in_specs=[pl.BlockSpec((emb, cb), lambda i: (0, i))],
        out_specs=pl.BlockSpec((q, 128), lambda i: (i, 0)),
        out_shape=jax.ShapeDtypeStruct((nblk * q, 128), jnp.int32),
    )


@functools.lru_cache(maxsize=None)
def _make_sc_pool(vocab_pad, emb, batch, seqlen, nc, ns):
    nw = nc * ns
    assert batch % nw == 0 and seqlen % 2 == 0 and emb == 64
    s_per = batch // nw
    half = seqlen // 2
    rows_per = s_per * 2
    words = emb // 2
    unroll = 10
    assert half % unroll == 0
    mesh = plsc.VectorSubcoreMesh(core_axis_name="c", subcore_axis_name="s")

    @functools.partial(
        pl.kernel,
        out_type=jax.ShapeDtypeStruct((batch, emb), jnp.float32),
        mesh=mesh,
        scratch_types=[
            pltpu.VMEM((rows_per, half), jnp.int32),
            pltpu.VMEM((8, half, words), jnp.int32),
            pltpu.VMEM((s_per, emb), jnp.float32),
        ] + [pltpu.SemaphoreType.DMA] * 8,
        compiler_params=pltpu.CompilerParams(use_tc_tiling_on_sc=False,
                                             needs_layout_passes=False),
    )
    def sc_pool(idx_hbm, table_hbm, sums_hbm, idx_v, rows_v, sums_v, *sems):
        wid = lax.axis_index("s") * nc + lax.axis_index("c")
        row_base = wid * rows_per
        pltpu.sync_copy(idx_hbm.at[pl.ds(row_base, rows_per)], idx_v)
        nring = len(sems)
        slots = [rows_v.at[r] for r in range(nring)]

        def fire(j, dst, sem):
            pltpu.async_copy(table_hbm.at[idx_v.at[j]], dst, sem)

        def wait(j, dst, sem):
            pltpu.make_async_copy(table_hbm.at[idx_v.at[j]], dst, sem).wait()

        def accum(rows, acc):
            def body(i, carry):
                carry = list(carry)
                r = i * unroll
                for k in range(unroll):
                    off = (k % 2) * 4
                    w0 = rows[r + k, pl.ds(0, _LANES)]
                    w1 = rows[r + k, pl.ds(_LANES, _LANES)]
                    carry[off + 0] += plsc.bitcast(w0 & _HI, jnp.float32)
                    carry[off + 1] += plsc.bitcast(w1 & _HI, jnp.float32)
                    carry[off + 2] += plsc.bitcast(
                        lax.shift_left(w0, 16), jnp.float32)
                    carry[off + 3] += plsc.bitcast(
                        lax.shift_left(w1, 16), jnp.float32)
                return tuple(carry)

            return lax.fori_loop(0, half // unroll, body, acc)

        zero = jnp.zeros((_LANES,), jnp.float32)
        for r in range(nring):
            fire(r, slots[r], sems[r])

        def store(s, acc):
            sums_v[s, pl.ds(0, _LANES)] = acc[0] + acc[4]
            sums_v[s, pl.ds(_LANES, _LANES)] = acc[1] + acc[5]
            sums_v[s, pl.ds(2 * _LANES, _LANES)] = acc[2] + acc[6]
            sums_v[s, pl.ds(3 * _LANES, _LANES)] = acc[3] + acc[7]

        def group(i, carry):
            j = nring * i
            acc = None
            for r in range(nring):
                wait(j + r, slots[r], sems[r])
                acc = accum(slots[r], acc if r % 2 else (zero,) * 8)

                @pl.when(i < 2 * s_per // nring - 1)
                def _():
                    fire(j + r + nring, slots[r], sems[r])

                if r % 2:
                    store((nring // 2) * i + r // 2, acc)
            return carry

        lax.fori_loop(0, 2 * s_per // nring, group, 0)
        pltpu.sync_copy(sums_v, sums_hbm.at[pl.ds(wid * s_per, s_per)])

    return sc_pool


@functools.lru_cache(maxsize=None)
def _make_tc_head(batch, seqlen, emb, nclass, pad):
    bb = 512
    assert batch % bb == 0

    def body(sent_ref, sums_ref, w_ref, b_ref, out_ref):
        cnt = jnp.sum((sent_ref[...] != pad).astype(jnp.float32), axis=1,
                      keepdims=True)
        pooled = sums_ref[...] / cnt
        logits = lax.dot_general(pooled, w_ref[...], (((1,), (1,)), ((), ())),
                                 preferred_element_type=jnp.float32)
        logits = logits + b_ref[...]
        shifted = logits - jnp.max(logits, axis=1, keepdims=True)
        lse = jnp.log(jnp.sum(jnp.exp(shifted), axis=1, keepdims=True))
        out_ref[...] = shifted - lse

    return pl.pallas_call(
        body,
        grid=(batch // bb,),
        in_specs=[
            pl.BlockSpec((bb, seqlen), lambda i: (i, 0)),
            pl.BlockSpec((bb, emb), lambda i: (i, 0)),
            pl.BlockSpec((nclass, emb), lambda i: (0, 0)),
            pl.BlockSpec((1, nclass), lambda i: (0, 0)),
        ],
        out_specs=pl.BlockSpec((bb, nclass), lambda i: (i, 0)),
        out_shape=jax.ShapeDtypeStruct((batch, nclass), jnp.float32),
    )


def kernel(sentences, emb_table, fc_w, fc_b):
    batch, seqlen = sentences.shape
    vocab, emb = emb_table.shape
    nclass = fc_w.shape[0]
    info = plsc.get_sparse_core_info()
    nc, ns = info.num_cores, info.num_subcores
    sent_i32 = sentences.astype(jnp.int32)
    cb = 16384
    q = cb // 4
    nblk = (vocab + cb - 1) // cb
    vocab_pad = nblk * cb
    packed = _make_tc_packer(vocab, emb, cb)(emb_table.T)
    table_lin = packed.reshape(vocab_pad, emb // 2)
    lo = sent_i32 % cb
    idx_t = (sent_i32 - lo) + 4 * (lo % q) + lo // q
    idx_flat = lax.optimization_barrier(
        idx_t.reshape(batch * seqlen // 128, 128))
    idx2 = idx_flat.reshape(batch * 2, seqlen // 2)
    sums = _make_sc_pool(vocab_pad, emb, batch, seqlen, nc, ns)(idx2, table_lin)
    head = _make_tc_head(batch, seqlen, emb, nclass, 0)
    return head(sent_i32, sums, fc_w, fc_b.reshape(1, nclass))


# speedup vs baseline: 3.4918x; 1.0460x over previous
import functools

import jax
import jax.numpy as jnp
from jax import lax
from jax.experimental import pallas as pl
from jax.experimental.pallas import tpu as pltpu
from jax.experimental.pallas import tpu_sc as plsc

_LANES = 16
_HI = -65536


def _round_bf16_hi(x):
    b = lax.bitcast_convert_type(x, jnp.int32)
    return b + 0x7FFF + (lax.shift_right_logical(b, 16) & 1)


@functools.lru_cache(maxsize=None)
def _make_tc_packer(vocab, emb, cb):
    assert emb == 64
    nblk = (vocab + cb - 1) // cb
    q = cb // 4

    def body(t_ref, out_ref):
        x = t_ref[...]
        hi = _round_bf16_hi(x[0:32, :]) & _HI
        lo = lax.shift_right_logical(_round_bf16_hi(x[32:64, :]), 16)
        w = hi | lo
        wp4 = jnp.concatenate(
            [w[:, 0:q], w[:, q:2 * q], w[:, 2 * q:3 * q], w[:, 3 * q:4 * q]],
            axis=0)
        out_ref[...] = wp4.T

    return pl.pallas_call(
        body,
        grid=(nblk,),
        in_specs=[pl.BlockSpec((emb, cb), lambda i: (0, i))],
        out_specs=pl.BlockSpec((q, 128), lambda i: (i, 0)),
        out_shape=jax.ShapeDtypeStruct((nblk * q, 128), jnp.int32),
    )


@functools.lru_cache(maxsize=None)
def _make_sc_pool(vocab_pad, emb, batch, seqlen, nc, ns):
    nw = nc * ns
    assert batch % nw == 0 and emb == 64
    s_per = batch // nw
    tokens = s_per * seqlen
    assert tokens % 128 == 0
    nchunk = tokens // 128
    words = emb // 2
    import math
    pat_tok = seqlen * 128 // math.gcd(seqlen, 128)
    pat = pat_tok // 128
    pat_sent = pat_tok // seqlen
    reps = tokens // pat_tok
    nring = 5
    assert pat % nring == 0
    mesh = plsc.VectorSubcoreMesh(core_axis_name="c", subcore_axis_name="s")

    @functools.partial(
        pl.kernel,
        out_type=jax.ShapeDtypeStruct((batch, emb), jnp.float32),
        mesh=mesh,
        scratch_types=[
            pltpu.VMEM((nchunk, 128), jnp.int32),
            pltpu.VMEM((nring, 128, words), jnp.int32),
            pltpu.VMEM((s_per, emb), jnp.float32),
        ] + [pltpu.SemaphoreType.DMA] * nring,
        compiler_params=pltpu.CompilerParams(use_tc_tiling_on_sc=False,
                                             needs_layout_passes=False),
    )
    def sc_pool(idx_hbm, table_hbm, sums_hbm, idx_v, rows_v, sums_v, *sems):
        wid = lax.axis_index("s") * nc + lax.axis_index("c")
        pltpu.sync_copy(idx_hbm.at[pl.ds(wid * nchunk, nchunk)], idx_v)
        slots = [rows_v.at[r] for r in range(nring)]

        def fire(j, dst, sem):
            pltpu.async_copy(table_hbm.at[idx_v.at[j]], dst, sem)

        def wait(j, dst, sem):
            pltpu.make_async_copy(table_hbm.at[idx_v.at[j]], dst, sem).wait()

        def accum_range(rows, a, b, acc):
            def body(i, carry):
                carry = list(carry)
                r = a + i * 8
                for k in range(8):
                    off = (k % 2) * 4
                    w0 = rows[r + k, pl.ds(0, _LANES)]
                    w1 = rows[r + k, pl.ds(_LANES, _LANES)]
                    carry[off + 0] += plsc.bitcast(w0 & _HI, jnp.float32)
                    carry[off + 1] += plsc.bitcast(w1 & _HI, jnp.float32)
                    carry[off + 2] += plsc.bitcast(
                        lax.shift_left(w0, 16), jnp.float32)
                    carry[off + 3] += plsc.bitcast(
                        lax.shift_left(w1, 16), jnp.float32)
                return tuple(carry)

            return lax.fori_loop(0, (b - a) // 8, body, acc)

        zero8 = (jnp.zeros((_LANES,), jnp.float32),) * 8
        for r in range(nring):
            fire(r, slots[r], sems[r])

        def store(s, acc):
            sums_v[s, pl.ds(0, _LANES)] = acc[0] + acc[4]
            sums_v[s, pl.ds(_LANES, _LANES)] = acc[1] + acc[5]
            sums_v[s, pl.ds(2 * _LANES, _LANES)] = acc[2] + acc[6]
            sums_v[s, pl.ds(3 * _LANES, _LANES)] = acc[3] + acc[7]

        def rep(i, carry):
            jb = pat * i
            sb = pat_sent * i
            acc = zero8
            for r in range(pat):
                slot = r % nring
                wait(jb + r, slots[slot], sems[slot])
                start = 128 * r
                cut = seqlen - start % seqlen
                if cut <= 128:
                    acc = accum_range(slots[slot], 0, cut, acc)
                    store(sb + start // seqlen, acc)
                    acc = zero8
                    if cut < 128:
                        acc = accum_range(slots[slot], cut, 128, acc)
                else:
                    acc = accum_range(slots[slot], 0, 128, acc)
                if r < pat - nring:
                    fire(jb + r + nring, slots[slot], sems[slot])
                else:
                    @pl.when(i < reps - 1)
                    def _():
                        fire(jb + r + nring, slots[slot], sems[slot])
            return carry

        lax.fori_loop(0, reps, rep, 0)
        pltpu.sync_copy(sums_v, sums_hbm.at[pl.ds(wid * s_per, s_per)])

    return sc_pool


@functools.lru_cache(maxsize=None)
def _make_tc_head(batch, seqlen, emb, nclass, pad):
    bb = 512
    assert batch % bb == 0

    def body(sent_ref, sums_ref, w_ref, b_ref, out_ref):
        cnt = jnp.sum((sent_ref[...] != pad).astype(jnp.float32), axis=1,
                      keepdims=True)
        pooled = sums_ref[...] / cnt
        logits = lax.dot_general(pooled, w_ref[...], (((1,), (1,)), ((), ())),
                                 preferred_element_type=jnp.float32)
        logits = logits + b_ref[...]
        shifted = logits - jnp.max(logits, axis=1, keepdims=True)
        lse = jnp.log(jnp.sum(jnp.exp(shifted), axis=1, keepdims=True))
        out_ref[...] = shifted - lse

    return pl.pallas_call(
        body,
        grid=(batch // bb,),
        in_specs=[
            pl.BlockSpec((bb, seqlen), lambda i: (i, 0)),
            pl.BlockSpec((bb, emb), lambda i: (i, 0)),
            pl.BlockSpec((nclass, emb), lambda i: (0, 0)),
            pl.BlockSpec((1, nclass), lambda i: (0, 0)),
        ],
        out_specs=pl.BlockSpec((bb, nclass), lambda i: (i, 0)),
        out_shape=jax.ShapeDtypeStruct((batch, nclass), jnp.float32),
    )


def kernel(sentences, emb_table, fc_w, fc_b):
    batch, seqlen = sentences.shape
    vocab, emb = emb_table.shape
    nclass = fc_w.shape[0]
    info = plsc.get_sparse_core_info()
    nc, ns = info.num_cores, info.num_subcores
    sent_i32 = sentences.astype(jnp.int32)
    cb = 16384
    q = cb // 4
    nblk = (vocab + cb - 1) // cb
    vocab_pad = nblk * cb
    packed = _make_tc_packer(vocab, emb, cb)(emb_table.T)
    table_lin = packed.reshape(vocab_pad, emb // 2)
    lo = sent_i32 % cb
    idx_t = (sent_i32 - lo) + 4 * (lo % q) + lo // q
    idx_flat = lax.optimization_barrier(
        idx_t.reshape(batch * seqlen // 128, 128))
    sums = _make_sc_pool(vocab_pad, emb, batch, seqlen, nc, ns)(
        idx_flat, table_lin)
    head = _make_tc_head(batch, seqlen, emb, nclass, 0)
    return head(sent_i32, sums, fc_w, fc_b.reshape(1, nclass))


# speedup vs baseline: 3.4964x; 1.0013x over previous
import functools

import jax
import jax.numpy as jnp
from jax import lax
from jax.experimental import pallas as pl
from jax.experimental.pallas import tpu as pltpu
from jax.experimental.pallas import tpu_sc as plsc

_LANES = 16
_HI = -65536


def _round_bf16_hi(x):
    b = lax.bitcast_convert_type(x, jnp.int32)
    return b + 0x7FFF + (lax.shift_right_logical(b, 16) & 1)


@functools.lru_cache(maxsize=None)
def _make_tc_packer(vocab, emb, cb):
    assert emb == 64
    nblk = (vocab + cb - 1) // cb
    q = cb // 4

    def body(t_ref, out_ref):
        x = t_ref[...]
        hi = _round_bf16_hi(x[0:32, :]) & _HI
        lo = lax.shift_right_logical(_round_bf16_hi(x[32:64, :]), 16)
        w = hi | lo
        wp4 = jnp.concatenate(
            [w[:, 0:q], w[:, q:2 * q], w[:, 2 * q:3 * q], w[:, 3 * q:4 * q]],
            axis=0)
        out_ref[...] = wp4.T

    return pl.pallas_call(
        body,
        grid=(nblk,),
        in_specs=[pl.BlockSpec((emb, cb), lambda i: (0, i))],
        out_specs=pl.BlockSpec((q, 128), lambda i: (i, 0)),
        out_shape=jax.ShapeDtypeStruct((nblk * q, 128), jnp.int32),
    )


@functools.lru_cache(maxsize=None)
def _make_sc_pool(vocab_pad, emb, batch, seqlen, nc, ns):
    nw = nc * ns
    assert batch % nw == 0 and emb == 64
    s_per = batch // nw
    tokens = s_per * seqlen
    assert tokens % 128 == 0
    nchunk = tokens // 128
    words = emb // 2
    import math
    pat_tok = seqlen * 128 // math.gcd(seqlen, 128)
    pat = pat_tok // 128
    pat_sent = pat_tok // seqlen
    reps = tokens // pat_tok
    nring = 10
    per_iter = 2
    assert (per_iter * pat) % nring == 0 and reps % per_iter == 0
    mesh = plsc.VectorSubcoreMesh(core_axis_name="c", subcore_axis_name="s")

    @functools.partial(
        pl.kernel,
        out_type=jax.ShapeDtypeStruct((batch, emb), jnp.float32),
        mesh=mesh,
        scratch_types=[
            pltpu.VMEM((nchunk, 128), jnp.int32),
            pltpu.VMEM((nring, 128, words), jnp.int32),
            pltpu.VMEM((s_per, emb), jnp.float32),
        ] + [pltpu.SemaphoreType.DMA] * nring,
        compiler_params=pltpu.CompilerParams(use_tc_tiling_on_sc=False,
                                             needs_layout_passes=False),
    )
    def sc_pool(idx_hbm, table_hbm, sums_hbm, idx_v, rows_v, sums_v, *sems):
        wid = lax.axis_index("s") * nc + lax.axis_index("c")
        pltpu.sync_copy(idx_hbm.at[pl.ds(wid * nchunk, nchunk)], idx_v)
        slots = [rows_v.at[r] for r in range(nring)]

        def fire(j, dst, sem):
            pltpu.async_copy(table_hbm.at[idx_v.at[j]], dst, sem)

        def wait(j, dst, sem):
            pltpu.make_async_copy(table_hbm.at[idx_v.at[j]], dst, sem).wait()

        def accum_range(rows, a, b, acc):
            def body(i, carry):
                carry = list(carry)
                r = a + i * 8
                for k in range(8):
                    off = (k % 2) * 4
                    w0 = rows[r + k, pl.ds(0, _LANES)]
                    w1 = rows[r + k, pl.ds(_LANES, _LANES)]
                    carry[off + 0] += plsc.bitcast(w0 & _HI, jnp.float32)
                    carry[off + 1] += plsc.bitcast(w1 & _HI, jnp.float32)
                    carry[off + 2] += plsc.bitcast(
                        lax.shift_left(w0, 16), jnp.float32)
                    carry[off + 3] += plsc.bitcast(
                        lax.shift_left(w1, 16), jnp.float32)
                return tuple(carry)

            return lax.fori_loop(0, (b - a) // 8, body, acc)

        zero8 = (jnp.zeros((_LANES,), jnp.float32),) * 8
        for r in range(nring):
            fire(r, slots[r], sems[r])

        def store(s, acc):
            sums_v[s, pl.ds(0, _LANES)] = acc[0] + acc[4]
            sums_v[s, pl.ds(_LANES, _LANES)] = acc[1] + acc[5]
            sums_v[s, pl.ds(2 * _LANES, _LANES)] = acc[2] + acc[6]
            sums_v[s, pl.ds(3 * _LANES, _LANES)] = acc[3] + acc[7]

        npat = per_iter * pat

        def rep(i, carry):
            jb = npat * i
            sb = per_iter * pat_sent * i
            acc = zero8
            for rr in range(npat):
                slot = rr % nring
                wait(jb + rr, slots[slot], sems[slot])
                r = rr % pat
                sb2 = sb + pat_sent * (rr // pat)
                start = 128 * r
                cut = seqlen - start % seqlen
                if cut <= 128:
                    acc = accum_range(slots[slot], 0, cut, acc)
                    store(sb2 + start // seqlen, acc)
                    acc = zero8
                    if cut < 128:
                        acc = accum_range(slots[slot], cut, 128, acc)
                else:
                    acc = accum_range(slots[slot], 0, 128, acc)
                if rr < npat - nring:
                    fire(jb + rr + nring, slots[slot], sems[slot])
                else:
                    @pl.when(i < reps // per_iter - 1)
                    def _():
                        fire(jb + rr + nring, slots[slot], sems[slot])
            return carry

        lax.fori_loop(0, reps // per_iter, rep, 0)
        pltpu.sync_copy(sums_v, sums_hbm.at[pl.ds(wid * s_per, s_per)])

    return sc_pool


@functools.lru_cache(maxsize=None)
def _make_tc_head(batch, seqlen, emb, nclass, pad):
    bb = 512
    assert batch % bb == 0

    def body(sent_ref, sums_ref, w_ref, b_ref, out_ref):
        cnt = jnp.sum((sent_ref[...] != pad).astype(jnp.float32), axis=1,
                      keepdims=True)
        pooled = sums_ref[...] / cnt
        logits = lax.dot_general(pooled, w_ref[...], (((1,), (1,)), ((), ())),
                                 preferred_element_type=jnp.float32)
        logits = logits + b_ref[...]
        shifted = logits - jnp.max(logits, axis=1, keepdims=True)
        lse = jnp.log(jnp.sum(jnp.exp(shifted), axis=1, keepdims=True))
        out_ref[...] = shifted - lse

    return pl.pallas_call(
        body,
        grid=(batch // bb,),
        in_specs=[
            pl.BlockSpec((bb, seqlen), lambda i: (i, 0)),
            pl.BlockSpec((bb, emb), lambda i: (i, 0)),
            pl.BlockSpec((nclass, emb), lambda i: (0, 0)),
            pl.BlockSpec((1, nclass), lambda i: (0, 0)),
        ],
        out_specs=pl.BlockSpec((bb, nclass), lambda i: (i, 0)),
        out_shape=jax.ShapeDtypeStruct((batch, nclass), jnp.float32),
    )


def kernel(sentences, emb_table, fc_w, fc_b):
    batch, seqlen = sentences.shape
    vocab, emb = emb_table.shape
    nclass = fc_w.shape[0]
    info = plsc.get_sparse_core_info()
    nc, ns = info.num_cores, info.num_subcores
    sent_i32 = sentences.astype(jnp.int32)
    cb = 16384
    q = cb // 4
    nblk = (vocab + cb - 1) // cb
    vocab_pad = nblk * cb
    packed = _make_tc_packer(vocab, emb, cb)(emb_table.T)
    table_lin = packed.reshape(vocab_pad, emb // 2)
    lo = sent_i32 % cb
    idx_t = (sent_i32 - lo) + 4 * (lo % q) + lo // q
    idx_flat = lax.optimization_barrier(
        idx_t.reshape(batch * seqlen // 128, 128))
    sums = _make_sc_pool(vocab_pad, emb, batch, seqlen, nc, ns)(
        idx_flat, table_lin)
    head = _make_tc_head(batch, seqlen, emb, nclass, 0)
    return head(sent_i32, sums, fc_w, fc_b.reshape(1, nclass))


# speedup vs baseline: 3.7073x; 1.0603x over previous
import functools

import jax
import jax.numpy as jnp
from jax import lax
from jax.experimental import pallas as pl
from jax.experimental.pallas import tpu as pltpu
from jax.experimental.pallas import tpu_sc as plsc

_LANES = 16
_HI = -65536


def _round_bf16_hi(x):
    b = lax.bitcast_convert_type(x, jnp.int32)
    return b + 0x7FFF + (lax.shift_right_logical(b, 16) & 1)


@functools.lru_cache(maxsize=None)
def _make_tc_packer(vocab, emb, cb):
    assert emb == 64
    nblk = (vocab + cb - 1) // cb
    q = cb // 4

    def body(t_ref, out_ref):
        x = t_ref[...]
        hi = _round_bf16_hi(x[0:32, :]) & _HI
        lo = lax.shift_right_logical(_round_bf16_hi(x[32:64, :]), 16)
        w = hi | lo
        wp4 = jnp.concatenate(
            [w[:, 0:q], w[:, q:2 * q], w[:, 2 * q:3 * q], w[:, 3 * q:4 * q]],
            axis=0)
        out_ref[...] = wp4.T

    return pl.pallas_call(
        body,
        grid=(nblk,),
        in_specs=[pl.BlockSpec((emb, cb), lambda i: (0, i))],
        out_specs=pl.BlockSpec((q, 128), lambda i: (i, 0)),
        out_shape=jax.ShapeDtypeStruct((nblk * q, 128), jnp.int32),
    )


@functools.lru_cache(maxsize=None)
def _make_sc_pool(vocab_pad, emb, batch, seqlen, nc, ns):
    nw = nc * ns
    assert batch % nw == 0 and emb == 64
    s_per = batch // nw
    tokens = s_per * seqlen
    assert tokens % 128 == 0
    nchunk = tokens // 128
    words = emb // 2
    import math
    pat_tok = seqlen * 128 // math.gcd(seqlen, 128)
    pat = pat_tok // 128
    pat_sent = pat_tok // seqlen
    reps = tokens // pat_tok
    nring = 10
    per_iter = 2
    assert (per_iter * pat) % nring == 0 and reps % per_iter == 0
    mesh = plsc.VectorSubcoreMesh(core_axis_name="c", subcore_axis_name="s")

    @functools.partial(
        pl.kernel,
        out_type=jax.ShapeDtypeStruct((batch, emb), jnp.float32),
        mesh=mesh,
        scratch_types=[
            pltpu.VMEM((nchunk, 128), jnp.int32),
            pltpu.VMEM((nring, 128, words), jnp.int32),
            pltpu.VMEM((s_per, emb), jnp.float32),
        ] + [pltpu.SemaphoreType.DMA] * nring,
        compiler_params=pltpu.CompilerParams(use_tc_tiling_on_sc=False,
                                             needs_layout_passes=False),
    )
    def sc_pool(idx_hbm, table_hbm, sums_hbm, idx_v, rows_v, sums_v, *sems):
        wid = lax.axis_index("s") * nc + lax.axis_index("c")
        pltpu.sync_copy(idx_hbm.at[pl.ds(wid * nchunk, nchunk)], idx_v)
        slots = [rows_v.at[r] for r in range(nring)]

        def fire(j, dst, sem):
            pltpu.async_copy(table_hbm.at[idx_v.at[j]], dst, sem)

        def wait(j, dst, sem):
            pltpu.make_async_copy(table_hbm.at[idx_v.at[j]], dst, sem).wait()

        def accum_range(rows, a, b, acc):
            def body(i, carry):
                carry = list(carry)
                r = a + i * 8
                for k in range(8):
                    off = (k % 2) * 4
                    w0 = rows[r + k, pl.ds(0, _LANES)]
                    w1 = rows[r + k, pl.ds(_LANES, _LANES)]
                    carry[off + 0] += plsc.bitcast(w0, jnp.float32)
                    carry[off + 1] += plsc.bitcast(w1, jnp.float32)
                    carry[off + 2] += plsc.bitcast(
                        lax.shift_left(w0, 16), jnp.float32)
                    carry[off + 3] += plsc.bitcast(
                        lax.shift_left(w1, 16), jnp.float32)
                return tuple(carry)

            return lax.fori_loop(0, (b - a) // 8, body, acc)

        zero8 = (jnp.zeros((_LANES,), jnp.float32),) * 8
        for r in range(nring):
            fire(r, slots[r], sems[r])

        def store(s, acc):
            sums_v[s, pl.ds(0, _LANES)] = acc[0] + acc[4]
            sums_v[s, pl.ds(_LANES, _LANES)] = acc[1] + acc[5]
            sums_v[s, pl.ds(2 * _LANES, _LANES)] = acc[2] + acc[6]
            sums_v[s, pl.ds(3 * _LANES, _LANES)] = acc[3] + acc[7]

        npat = per_iter * pat

        def rep(i, carry):
            jb = npat * i
            sb = per_iter * pat_sent * i
            acc = zero8
            for rr in range(npat):
                slot = rr % nring
                wait(jb + rr, slots[slot], sems[slot])
                r = rr % pat
                sb2 = sb + pat_sent * (rr // pat)
                start = 128 * r
                cut = seqlen - start % seqlen
                if cut <= 128:
                    acc = accum_range(slots[slot], 0, cut, acc)
                    store(sb2 + start // seqlen, acc)
                    acc = zero8
                    if cut < 128:
                        acc = accum_range(slots[slot], cut, 128, acc)
                else:
                    acc = accum_range(slots[slot], 0, 128, acc)
                if rr < npat - nring:
                    fire(jb + rr + nring, slots[slot], sems[slot])
                else:
                    @pl.when(i < reps // per_iter - 1)
                    def _():
                        fire(jb + rr + nring, slots[slot], sems[slot])
            return carry

        lax.fori_loop(0, reps // per_iter, rep, 0)
        pltpu.sync_copy(sums_v, sums_hbm.at[pl.ds(wid * s_per, s_per)])

    return sc_pool


@functools.lru_cache(maxsize=None)
def _make_tc_head(batch, seqlen, emb, nclass, pad):
    bb = 512
    assert batch % bb == 0

    def body(sent_ref, sums_ref, w_ref, b_ref, out_ref):
        cnt = jnp.sum((sent_ref[...] != pad).astype(jnp.float32), axis=1,
                      keepdims=True)
        pooled = sums_ref[...] / cnt
        logits = lax.dot_general(pooled, w_ref[...], (((1,), (1,)), ((), ())),
                                 preferred_element_type=jnp.float32)
        logits = logits + b_ref[...]
        shifted = logits - jnp.max(logits, axis=1, keepdims=True)
        lse = jnp.log(jnp.sum(jnp.exp(shifted), axis=1, keepdims=True))
        out_ref[...] = shifted - lse

    return pl.pallas_call(
        body,
        grid=(batch // bb,),
        in_specs=[
            pl.BlockSpec((bb, seqlen), lambda i: (i, 0)),
            pl.BlockSpec((bb, emb), lambda i: (i, 0)),
            pl.BlockSpec((nclass, emb), lambda i: (0, 0)),
            pl.BlockSpec((1, nclass), lambda i: (0, 0)),
        ],
        out_specs=pl.BlockSpec((bb, nclass), lambda i: (i, 0)),
        out_shape=jax.ShapeDtypeStruct((batch, nclass), jnp.float32),
    )


def kernel(sentences, emb_table, fc_w, fc_b):
    batch, seqlen = sentences.shape
    vocab, emb = emb_table.shape
    nclass = fc_w.shape[0]
    info = plsc.get_sparse_core_info()
    nc, ns = info.num_cores, info.num_subcores
    sent_i32 = sentences.astype(jnp.int32)
    cb = 16384
    q = cb // 4
    nblk = (vocab + cb - 1) // cb
    vocab_pad = nblk * cb
    packed = _make_tc_packer(vocab, emb, cb)(emb_table.T)
    table_lin = packed.reshape(vocab_pad, emb // 2)
    lo = sent_i32 % cb
    idx_t = (sent_i32 - lo) + 4 * (lo % q) + lo // q
    idx_flat = lax.optimization_barrier(
        idx_t.reshape(batch * seqlen // 128, 128))
    sums = _make_sc_pool(vocab_pad, emb, batch, seqlen, nc, ns)(
        idx_flat, table_lin)
    head = _make_tc_head(batch, seqlen, emb, nclass, 0)
    return head(sent_i32, sums, fc_w, fc_b.reshape(1, nclass))


# speedup vs baseline: 3.8082x; 1.0272x over previous
import functools

import jax
import jax.numpy as jnp
from jax import lax
from jax.experimental import pallas as pl
from jax.experimental.pallas import tpu as pltpu
from jax.experimental.pallas import tpu_sc as plsc

_LANES = 16
_HI = -65536


def _round_bf16_hi(x):
    b = lax.bitcast_convert_type(x, jnp.int32)
    return b + 0x7FFF + (lax.shift_right_logical(b, 16) & 1)


@functools.lru_cache(maxsize=None)
def _make_tc_packer(vocab, emb, cb):
    assert emb == 64
    nblk = (vocab + cb - 1) // cb
    q = cb // 4

    def body(t_ref, out_ref):
        x = t_ref[...]
        hi = _round_bf16_hi(x[0:32, :]) & _HI
        lo = lax.shift_right_logical(_round_bf16_hi(x[32:64, :]), 16)
        w = hi | lo
        wp4 = jnp.concatenate(
            [w[:, 0:q], w[:, q:2 * q], w[:, 2 * q:3 * q], w[:, 3 * q:4 * q]],
            axis=0)
        out_ref[...] = wp4.T

    return pl.pallas_call(
        body,
        grid=(nblk,),
        in_specs=[pl.BlockSpec((emb, cb), lambda i: (0, i))],
        out_specs=pl.BlockSpec((q, 128), lambda i: (i, 0)),
        out_shape=jax.ShapeDtypeStruct((nblk * q, 128), jnp.int32),
    )


@functools.lru_cache(maxsize=None)
def _make_sc_pool(vocab_pad, emb, batch, seqlen, nc, ns):
    nw = nc * ns
    assert batch % nw == 0 and emb == 64
    s_per = batch // nw
    tokens = s_per * seqlen
    assert tokens % 128 == 0
    nchunk = tokens // 128
    words = emb // 2
    import math
    pat_tok = seqlen * 128 // math.gcd(seqlen, 128)
    pat = pat_tok // 128
    pat_sent = pat_tok // seqlen
    reps = tokens // pat_tok
    nring = 10
    per_iter = 2
    assert (per_iter * pat) % nring == 0 and reps % per_iter == 0
    mesh = plsc.VectorSubcoreMesh(core_axis_name="c", subcore_axis_name="s")

    @functools.partial(
        pl.kernel,
        out_type=jax.ShapeDtypeStruct((batch, emb), jnp.float32),
        mesh=mesh,
        scratch_types=[
            pltpu.VMEM((nchunk, 128), jnp.int32),
            pltpu.VMEM((nring, 128, words), jnp.int32),
            pltpu.VMEM((s_per, emb), jnp.float32),
        ] + [pltpu.SemaphoreType.DMA] * nring,
        compiler_params=pltpu.CompilerParams(use_tc_tiling_on_sc=False,
                                             needs_layout_passes=False),
    )
    def sc_pool(idx_hbm, table_hbm, sums_hbm, idx_v, rows_v, sums_v, *sems):
        wid = lax.axis_index("s") * nc + lax.axis_index("c")
        pltpu.sync_copy(idx_hbm.at[pl.ds(wid * nchunk, nchunk)], idx_v)
        slots = [rows_v.at[r] for r in range(nring)]

        def fire(j, dst, sem):
            pltpu.async_copy(table_hbm.at[idx_v.at[j]], dst, sem)

        def wait(j, dst, sem):
            pltpu.make_async_copy(table_hbm.at[idx_v.at[j]], dst, sem).wait()

        def accum_range(rows, a, b, acc):
            def body(i, carry):
                carry = list(carry)
                r = a + i * 8
                for k in range(8):
                    off = (k % 2) * 4
                    w0 = rows[r + k, pl.ds(0, _LANES)]
                    w1 = rows[r + k, pl.ds(_LANES, _LANES)]
                    carry[off + 0] += plsc.bitcast(w0, jnp.float32)
                    carry[off + 1] += plsc.bitcast(w1, jnp.float32)
                    carry[off + 2] += plsc.bitcast(
                        lax.shift_left(w0, 16), jnp.float32)
                    carry[off + 3] += plsc.bitcast(
                        lax.shift_left(w1, 16), jnp.float32)
                return tuple(carry)

            return lax.fori_loop(0, (b - a) // 8, body, acc)

        zero8 = (jnp.zeros((_LANES,), jnp.float32),) * 8
        for r in range(nring):
            fire(r, slots[r], sems[r])

        def store(s, acc):
            sums_v[s, pl.ds(0, _LANES)] = acc[0] + acc[4]
            sums_v[s, pl.ds(_LANES, _LANES)] = acc[1] + acc[5]
            sums_v[s, pl.ds(2 * _LANES, _LANES)] = acc[2] + acc[6]
            sums_v[s, pl.ds(3 * _LANES, _LANES)] = acc[3] + acc[7]

        npat = per_iter * pat

        def rep(i, carry):
            jb = npat * i
            sb = per_iter * pat_sent * i
            acc = zero8
            for rr in range(npat):
                slot = rr % nring
                wait(jb + rr, slots[slot], sems[slot])
                r = rr % pat
                sb2 = sb + pat_sent * (rr // pat)
                start = 128 * r
                cut = seqlen - start % seqlen
                if cut <= 128:
                    acc = accum_range(slots[slot], 0, cut, acc)
                    store(sb2 + start // seqlen, acc)
                    acc = zero8
                    if cut < 128:
                        acc = accum_range(slots[slot], cut, 128, acc)
                else:
                    acc = accum_range(slots[slot], 0, 128, acc)
                if rr < npat - nring:
                    fire(jb + rr + nring, slots[slot], sems[slot])
                else:
                    @pl.when(i < reps // per_iter - 1)
                    def _():
                        fire(jb + rr + nring, slots[slot], sems[slot])
            return carry

        lax.fori_loop(0, reps // per_iter, rep, 0)
        pltpu.sync_copy(sums_v, sums_hbm.at[pl.ds(wid * s_per, s_per)])

    return sc_pool


@functools.lru_cache(maxsize=None)
def _make_tc_head(batch, seqlen, emb, nclass, pad):
    bb = 512
    assert batch % bb == 0

    def body(sent_ref, sums_ref, w_ref, b_ref, out_ref):
        cnt = jnp.sum((sent_ref[...] != pad).astype(jnp.float32), axis=1,
                      keepdims=True)
        pooled = sums_ref[...] / cnt
        logits = lax.dot_general(pooled, w_ref[...], (((1,), (1,)), ((), ())),
                                 preferred_element_type=jnp.float32)
        logits = logits + b_ref[...]
        shifted = logits - jnp.max(logits, axis=1, keepdims=True)
        lse = jnp.log(jnp.sum(jnp.exp(shifted), axis=1, keepdims=True))
        out_ref[...] = shifted - lse

    return pl.pallas_call(
        body,
        grid=(batch // bb,),
        in_specs=[
            pl.BlockSpec((bb, seqlen), lambda i: (i, 0)),
            pl.BlockSpec((bb, emb), lambda i: (i, 0)),
            pl.BlockSpec((nclass, emb), lambda i: (0, 0)),
            pl.BlockSpec((1, nclass), lambda i: (0, 0)),
        ],
        out_specs=pl.BlockSpec((bb, nclass), lambda i: (i, 0)),
        out_shape=jax.ShapeDtypeStruct((batch, nclass), jnp.float32),
    )


def kernel(sentences, emb_table, fc_w, fc_b):
    batch, seqlen = sentences.shape
    vocab, emb = emb_table.shape
    nclass = fc_w.shape[0]
    info = plsc.get_sparse_core_info()
    nc, ns = info.num_cores, info.num_subcores
    sent_i32 = sentences.astype(jnp.int32)
    cb = 32768
    q = cb // 4
    nblk = (vocab + cb - 1) // cb
    vocab_pad = nblk * cb
    packed = _make_tc_packer(vocab, emb, cb)(emb_table.T)
    table_lin = packed.reshape(vocab_pad, emb // 2)
    lo = sent_i32 % cb
    idx_t = (sent_i32 - lo) + 4 * (lo % q) + lo // q
    idx_flat = lax.optimization_barrier(
        idx_t.reshape(batch * seqlen // 128, 128))
    sums = _make_sc_pool(vocab_pad, emb, batch, seqlen, nc, ns)(
        idx_flat, table_lin)
    head = _make_tc_head(batch, seqlen, emb, nclass, 0)
    return head(sent_i32, sums, fc_w, fc_b.reshape(1, nclass))
